# baseline (speedup 1.0000x reference)
import jax
import jax.numpy as jnp
from jax.experimental import pallas as pl
from jax.experimental.pallas import tpu as pltpu


def _mlp_kernel(x_ref, w1_ref, b1_ref, w2_ref, b2_ref, o_ref):
    h = jnp.dot(x_ref[...].astype(jnp.bfloat16),
                w1_ref[...],
                preferred_element_type=jnp.float32)
    h = jnp.maximum((h + b1_ref[...]).astype(jnp.bfloat16), 0)
    o_ref[...] = (jnp.dot(h, w2_ref[...],
                          preferred_element_type=jnp.float32)
                  + b2_ref[...])


def kernel(x, W1, b1, gamma, beta, running_mean, running_var, W2, b2):
    B, L, D_in = x.shape
    D_hid = W1.shape[1]
    D_out = W2.shape[1]
    M = B * L
    TM = 1024
    x2 = x.reshape(M, D_in)

    scale = gamma * jax.lax.rsqrt(running_var + 1e-5)
    shift = beta - running_mean * scale
    W2f = (scale[:, None] * W2).astype(jnp.bfloat16)
    b2f = shift @ W2 + b2

    out = pl.pallas_call(
        _mlp_kernel,
        grid=(M // TM,),
        in_specs=[
            pl.BlockSpec((TM, D_in), lambda i: (i, 0)),
            pl.BlockSpec((D_in, D_hid), lambda i: (0, 0)),
            pl.BlockSpec((1, D_hid), lambda i: (0, 0)),
            pl.BlockSpec((D_hid, D_out), lambda i: (0, 0)),
            pl.BlockSpec((1, D_out), lambda i: (0, 0)),
        ],
        out_specs=pl.BlockSpec((TM, D_out), lambda i: (i, 0)),
        out_shape=jax.ShapeDtypeStruct((M, D_out), jnp.float32),
        compiler_params=pltpu.CompilerParams(
            dimension_semantics=("parallel",)),
    )(x2, W1.astype(jnp.bfloat16), b1.reshape(1, -1), W2f,
      b2f.reshape(1, -1))
    return out.reshape(B, L, D_out)


# speedup vs baseline: 1.3501x; 1.3501x over previous
import jax
import jax.numpy as jnp
from jax.experimental import pallas as pl
from jax.experimental.pallas import tpu as pltpu


def _mlp_kernel(x_ref, w1_ref, b1_ref, w2_ref, b2_ref, o_ref):
    h = jnp.dot(x_ref[...].astype(jnp.bfloat16),
                w1_ref[...],
                preferred_element_type=jnp.float32)
    h = jnp.maximum((h + b1_ref[...]).astype(jnp.bfloat16), 0)
    o_ref[...] = (jnp.dot(h, w2_ref[...],
                          preferred_element_type=jnp.float32)
                  + b2_ref[...])


def kernel(x, W1, b1, gamma, beta, running_mean, running_var, W2, b2):
    B, L, D_in = x.shape
    D_hid = W1.shape[1]
    D_out = W2.shape[1]
    M = B * L
    TM = 4096
    x2 = x.reshape(M, D_in)

    scale = gamma * jax.lax.rsqrt(running_var + 1e-5)
    shift = beta - running_mean * scale
    W2f = (scale[:, None] * W2).astype(jnp.bfloat16)
    b2f = shift @ W2 + b2

    out = pl.pallas_call(
        _mlp_kernel,
        grid=(M // TM,),
        in_specs=[
            pl.BlockSpec((TM, D_in), lambda i: (i, 0)),
            pl.BlockSpec((D_in, D_hid), lambda i: (0, 0)),
            pl.BlockSpec((1, D_hid), lambda i: (0, 0)),
            pl.BlockSpec((D_hid, D_out), lambda i: (0, 0)),
            pl.BlockSpec((1, D_out), lambda i: (0, 0)),
        ],
        out_specs=pl.BlockSpec((TM, D_out), lambda i: (i, 0)),
        out_shape=jax.ShapeDtypeStruct((M, D_out), jnp.float32),
        compiler_params=pltpu.CompilerParams(
            dimension_semantics=("parallel",)),
    )(x2, W1.astype(jnp.bfloat16), b1.reshape(1, -1), W2f,
      b2f.reshape(1, -1))
    return out.reshape(B, L, D_out)


# speedup vs baseline: 1.9190x; 1.4214x over previous
import jax
import jax.numpy as jnp
from jax.experimental import pallas as pl
from jax.experimental.pallas import tpu as pltpu


def _mlp_kernel(x_ref, w1_ref, b1_ref, w2_ref, b2_ref, o_ref):
    h = jnp.dot(x_ref[...].astype(jnp.bfloat16),
                w1_ref[...],
                preferred_element_type=jnp.float32)
    h = jnp.maximum((h + b1_ref[...]).astype(jnp.bfloat16), 0)
    out = jnp.dot(h, w2_ref[...], preferred_element_type=jnp.float32)
    o_ref[...] = out.T + b2_ref[...]


def kernel(x, W1, b1, gamma, beta, running_mean, running_var, W2, b2):
    B, L, D_in = x.shape
    D_hid = W1.shape[1]
    D_out = W2.shape[1]
    M = B * L
    TM = 4096
    x2 = x.reshape(M, D_in)

    scale = gamma * jax.lax.rsqrt(running_var + 1e-5)
    shift = beta - running_mean * scale
    W2f = (scale[:, None] * W2).astype(jnp.bfloat16)
    b2f = shift @ W2 + b2

    out_t = pl.pallas_call(
        _mlp_kernel,
        grid=(M // TM,),
        in_specs=[
            pl.BlockSpec((TM, D_in), lambda i: (i, 0)),
            pl.BlockSpec((D_in, D_hid), lambda i: (0, 0)),
            pl.BlockSpec((1, D_hid), lambda i: (0, 0)),
            pl.BlockSpec((D_hid, D_out), lambda i: (0, 0)),
            pl.BlockSpec((D_out, 1), lambda i: (0, 0)),
        ],
        out_specs=pl.BlockSpec((D_out, TM), lambda i: (0, i)),
        out_shape=jax.ShapeDtypeStruct((D_out, M), jnp.float32),
        compiler_params=pltpu.CompilerParams(
            dimension_semantics=("parallel",)),
    )(x2, W1.astype(jnp.bfloat16), b1.reshape(1, -1), W2f,
      b2f.reshape(-1, 1))
    return out_t.T.reshape(B, L, D_out)
